# baseline (speedup 1.0000x reference)
import functools

import jax
import jax.numpy as jnp
from jax import lax
from jax.experimental import pallas as pl
from jax.experimental.pallas import tpu as pltpu
from jax.experimental.pallas import tpu_sc as plsc

N_EMB = 512
E_DIM = 64
ALPHA = 0.001
EPS = 1e-8
KB = 8


def _prep_body(xf_ref, w_ref, rp_ref,
               w0n_ref, q1t_ref, q2a_ref, pisc_ref, idx_ref, sp_ref):
    n = xf_ref.shape[0]
    seg = 256
    f32 = jnp.float32
    xf = xf_ref[...]
    w = w_ref[...]
    ones_row = jnp.ones((1, E_DIM), f32)

    x2c = jnp.sum(xf * xf, axis=1, keepdims=True)
    w2c = jnp.sum(w * w, axis=1, keepdims=True)
    x2r = lax.dot_general(ones_row, xf * xf, (((1,), (1,)), ((), ())),
                          preferred_element_type=f32)
    w2r = lax.dot_general(ones_row, w * w, (((1,), (1,)), ((), ())),
                          preferred_element_type=f32)
    mm = lax.dot_general(xf, w, (((1,), (1,)), ((), ())),
                         preferred_element_type=f32)
    mmt = lax.dot_general(w, xf, (((1,), (1,)), ((), ())),
                          preferred_element_type=f32)
    d = (x2c + w2r) - 2.0 * mm
    dt = (w2c + x2r) - 2.0 * mmt

    mn = jnp.min(d, axis=1, keepdims=True)
    iota_k = lax.broadcasted_iota(jnp.int32, (n, N_EMB), 1)
    cand = jnp.where(d == mn, iota_k, jnp.int32(N_EMB))
    idxc = jnp.min(cand, axis=1, keepdims=True)
    idx_ref[...] = idxc
    onehot = (iota_k == idxc).astype(f32)
    counts = jnp.sum(onehot, axis=0, keepdims=True)
    p = counts * (1.0 / n)
    rp = rp_ref[...]
    kl = jnp.sum(p * (jnp.log(p + 1e-10) - jnp.log(rp + 1e-10)))
    mse = jnp.sum(mn) * (1.0 / (n * E_DIM))
    sp_ref[...] = jnp.full((1, 1), 0.0, f32) + (1.25 * mse + 1.0 * kl)

    mx = jnp.max(d, axis=1, keepdims=True)
    sh = d - mx
    lse = jnp.log(jnp.sum(jnp.exp(sh), axis=1, keepdims=True))
    q = sh - lse
    mxr = jnp.max(dt, axis=0, keepdims=True)
    sht = dt - mxr
    lser = jnp.log(jnp.sum(jnp.exp(sht), axis=0, keepdims=True))
    qt = sht - lser

    lane_f = lax.broadcasted_iota(jnp.int32, (1, n), 1) % seg
    w0 = jnp.where(lane_f < seg - 2, qt, 0.0)
    q1t = jnp.concatenate([qt[:, 1:], qt[:, :1]], axis=1)
    q2s = jnp.concatenate([q[2:], q[:2]], axis=0)

    qpt = jnp.where(lane_f < seg - 1, qt, 0.0)
    cp = lax.dot_general(qpt, q1t, (((1,), (1,)), ((), ())),
                         preferred_element_type=f32)
    pi = cp * (1.0 / (jnp.sum(cp) + EPS))

    r2row = lax.dot_general(jnp.ones((1, n), f32), q2s,
                            (((1,), (1,)), ((), ())),
                            preferred_element_type=f32)
    s0km = lax.dot_general(w0 * r2row, q1t, (((1,), (1,)), ((), ())),
                           preferred_element_type=f32)
    dkm = s0km + (N_EMB * ALPHA + EPS)
    pisc_ref[...] = pi / dkm

    bf16 = jnp.bfloat16
    onesc = jnp.ones((N_EMB, 1), f32)
    zeros7 = jnp.zeros((N_EMB, 7), f32)
    w0n_ref[...] = jnp.concatenate([-w0, onesc, zeros7], axis=1).astype(bf16)
    q1t_ref[...] = jnp.concatenate([q1t, onesc, zeros7], axis=1).astype(bf16)
    r2col = jnp.sum(q2s, axis=1, keepdims=True)
    top = jnp.concatenate([q2s, r2col, zeros7], axis=1)
    brow = lax.broadcasted_iota(jnp.int32, (8, N_EMB + 8), 1)
    bsub = lax.broadcasted_iota(jnp.int32, (8, N_EMB + 8), 0)
    bot = jnp.where(bsub == 0,
                    jnp.where(brow < N_EMB, -ALPHA,
                              jnp.where(brow == N_EMB, -N_EMB * ALPHA, 0.0)),
                    0.0)
    q2a_ref[...] = jnp.concatenate([top, bot], axis=0).astype(bf16)


def _h2_body(w0n_ref, pisc_ref, q1t_ref, q2a_ref, sp_ref, out_ref, acc_ref):
    i = pl.program_id(0)
    nb = pl.num_programs(0)
    f32 = jnp.float32

    @pl.when(i == 0)
    def _init():
        acc_ref[...] = jnp.zeros((1, N_EMB), f32)

    q1t = q1t_ref[...]
    q2a = q2a_ref[...]
    for kk in range(KB):
        wrow = w0n_ref[kk:kk + 1, :]
        bmat = q1t * wrow
        outm = lax.dot_general(bmat, q2a, (((1,), (0,)), ((), ())),
                               preferred_element_type=f32)
        ncs = outm[:, :N_EMB]
        scol = outm[:, N_EMB:N_EMB + 1]
        negd = scol - EPS
        lrcp = -jnp.log(negd)
        u2 = ncs + EPS * negd
        l2 = jnp.log(u2) + lrcp
        term = ncs * l2
        piscrow = pisc_ref[kk:kk + 1, :]
        acc_ref[...] += lax.dot_general(
            piscrow, term, (((1,), (0,)), ((), ())),
            preferred_element_type=f32)

    @pl.when(i == nb - 1)
    def _fin():
        out_ref[...] = sp_ref[...] + 0.1 * jnp.sum(acc_ref[...],
                                                   axis=1, keepdims=True)


def _sc_gather(weight, idx):
    dpad = 128
    info = plsc.get_sparse_core_info()
    nw = info.num_cores * info.num_subcores
    bpw = N_EMB // nw
    mesh = plsc.VectorSubcoreMesh(core_axis_name="c", subcore_axis_name="s")

    @functools.partial(
        pl.kernel, mesh=mesh,
        out_type=jax.ShapeDtypeStruct((N_EMB, dpad), jnp.float32),
        scratch_types=[
            pltpu.VMEM((bpw,), jnp.int32),
            pltpu.VMEM((bpw, dpad), jnp.float32),
            pltpu.SemaphoreType.DMA,
        ],
    )
    def gk(table_hbm, idx_hbm, out_hbm, idx_v, rows_v, sem):
        wid = lax.axis_index("s") * info.num_cores + lax.axis_index("c")
        base = wid * bpw
        pltpu.sync_copy(idx_hbm.at[pl.ds(base, bpw)], idx_v)
        pltpu.async_copy(table_hbm.at[idx_v], rows_v, sem).wait()
        pltpu.sync_copy(rows_v, out_hbm.at[pl.ds(base, bpw)])

    wp = jnp.pad(weight, ((0, 0), (0, dpad - E_DIM)))
    return gk(wp, idx)[:, :E_DIM]


def kernel(x, batch, weight, running_prior):
    n = x.shape[0] * x.shape[1]
    xf = x.reshape(n, E_DIM)
    rp = running_prior.reshape(1, N_EMB)

    f32 = jnp.float32
    fa = N_EMB + 8
    w0n, q1t, q2a, pisc, idx2d, sp = pl.pallas_call(
        _prep_body,
        out_shape=[
            jax.ShapeDtypeStruct((N_EMB, fa), jnp.bfloat16),
            jax.ShapeDtypeStruct((N_EMB, fa), jnp.bfloat16),
            jax.ShapeDtypeStruct((fa, fa), jnp.bfloat16),
            jax.ShapeDtypeStruct((N_EMB, N_EMB), f32),
            jax.ShapeDtypeStruct((n, 1), jnp.int32),
            jax.ShapeDtypeStruct((1, 1), f32),
        ],
    )(xf, weight, rp)

    quant = _sc_gather(weight, idx2d.reshape(n))

    nb = N_EMB // KB
    total = pl.pallas_call(
        _h2_body,
        grid=(nb,),
        in_specs=[
            pl.BlockSpec((KB, fa), lambda i: (i, 0)),
            pl.BlockSpec((KB, N_EMB), lambda i: (i, 0)),
            pl.BlockSpec((N_EMB, fa), lambda i: (0, 0)),
            pl.BlockSpec((fa, fa), lambda i: (0, 0)),
            pl.BlockSpec((1, 1), lambda i: (0, 0)),
        ],
        out_specs=pl.BlockSpec((1, 1), lambda i: (0, 0)),
        out_shape=jax.ShapeDtypeStruct((1, 1), f32),
        scratch_shapes=[pltpu.VMEM((1, N_EMB), f32)],
    )(w0n, pisc, q1t, q2a, sp)

    return quant.reshape(x.shape), total[0, 0]


# speedup vs baseline: 1.8566x; 1.8566x over previous
import functools

import jax
import jax.numpy as jnp
from jax import lax
from jax.experimental import pallas as pl
from jax.experimental.pallas import tpu as pltpu
from jax.experimental.pallas import tpu_sc as plsc

N_EMB = 512
E_DIM = 64
ALPHA = 0.001
EPS = 1e-8
KB = 8


def _prep_body(xf_ref, w_ref, rp_ref,
               w0n_ref, q1t_ref, q1r_ref, q2a_ref, pisc_ref, idx_ref, sp_ref):
    n = xf_ref.shape[0]
    seg = 256
    f32 = jnp.float32
    xf = xf_ref[...]
    w = w_ref[...]
    ones_row = jnp.ones((1, E_DIM), f32)

    x2c = jnp.sum(xf * xf, axis=1, keepdims=True)
    w2c = jnp.sum(w * w, axis=1, keepdims=True)
    x2r = lax.dot_general(ones_row, xf * xf, (((1,), (1,)), ((), ())),
                          preferred_element_type=f32)
    w2r = lax.dot_general(ones_row, w * w, (((1,), (1,)), ((), ())),
                          preferred_element_type=f32)
    mm = lax.dot_general(xf, w, (((1,), (1,)), ((), ())),
                         preferred_element_type=f32)
    mmt = lax.dot_general(w, xf, (((1,), (1,)), ((), ())),
                          preferred_element_type=f32)
    d = (x2c + w2r) - 2.0 * mm
    dt = (w2c + x2r) - 2.0 * mmt

    mn = jnp.min(d, axis=1, keepdims=True)
    iota_k = lax.broadcasted_iota(jnp.int32, (n, N_EMB), 1)
    cand = jnp.where(d == mn, iota_k, jnp.int32(N_EMB))
    idxc = jnp.min(cand, axis=1, keepdims=True)
    idx_ref[...] = idxc
    onehot = (iota_k == idxc).astype(f32)
    counts = jnp.sum(onehot, axis=0, keepdims=True)
    p = counts * (1.0 / n)
    rp = rp_ref[...]
    kl = jnp.sum(p * (jnp.log(p + 1e-10) - jnp.log(rp + 1e-10)))
    mse = jnp.sum(mn) * (1.0 / (n * E_DIM))
    sp_ref[...] = jnp.full((1, 1), 0.0, f32) + (1.25 * mse + 1.0 * kl)

    mx = jnp.max(d, axis=1, keepdims=True)
    sh = d - mx
    lse = jnp.log(jnp.sum(jnp.exp(sh), axis=1, keepdims=True))
    q = sh - lse
    mxr = jnp.max(dt, axis=0, keepdims=True)
    sht = dt - mxr
    lser = jnp.log(jnp.sum(jnp.exp(sht), axis=0, keepdims=True))
    qt = sht - lser

    lane_f = lax.broadcasted_iota(jnp.int32, (1, n), 1) % seg
    w0 = jnp.where(lane_f < seg - 2, qt, 0.0)
    q1t = jnp.concatenate([qt[:, 1:], qt[:, :1]], axis=1)
    q2s = jnp.concatenate([q[2:], q[:2]], axis=0)

    qpt = jnp.where(lane_f < seg - 1, qt, 0.0)
    cp = lax.dot_general(qpt, q1t, (((1,), (1,)), ((), ())),
                         preferred_element_type=f32)
    pi = cp * (1.0 / (jnp.sum(cp) + EPS))

    r2row = lax.dot_general(jnp.ones((1, n), f32), q2s,
                            (((1,), (1,)), ((), ())),
                            preferred_element_type=f32)
    s0km = lax.dot_general(w0 * r2row, q1t, (((1,), (1,)), ((), ())),
                           preferred_element_type=f32)
    dkm = s0km + (N_EMB * ALPHA + EPS)
    pisc_ref[...] = pi / dkm

    bf16 = jnp.bfloat16
    w0n_ref[...] = (-w0).astype(bf16)
    q1t_ref[...] = q1t.astype(bf16)
    q1r_ref[...] = (q1t * r2row).astype(bf16)
    q2a_ref[...] = q2s.astype(bf16)


def _h2_body(w0n_ref, pisc_ref, q1t_ref, q1r_ref, q2a_ref, sp_ref,
             out_ref, acc_ref):
    i = pl.program_id(0)
    nb = pl.num_programs(0)
    f32 = jnp.float32

    @pl.when(i == 0)
    def _init():
        acc_ref[...] = jnp.zeros((1, N_EMB), f32)

    q1t = q1t_ref[...]
    q2a = q2a_ref[...]
    w0n = w0n_ref[...]
    s0blk = lax.dot_general(q1r_ref[...], w0n, (((1,), (1,)), ((), ())),
                            preferred_element_type=f32)
    negdall = s0blk - (N_EMB * ALPHA + EPS)
    lrcpall = -jnp.log(negdall)
    ucolall = EPS * negdall - ALPHA
    for kk in range(KB):
        wrow = w0n[kk:kk + 1, :]
        bmat = q1t * wrow
        ncs = lax.dot_general(bmat, q2a, (((1,), (0,)), ((), ())),
                              preferred_element_type=f32)
        ucol = ucolall[:, kk:kk + 1]
        lrcp = lrcpall[:, kk:kk + 1]
        u2 = ncs + ucol
        l2 = jnp.log(u2) + lrcp
        term = ncs * l2
        piscrow = pisc_ref[kk:kk + 1, :]
        acc_ref[...] += lax.dot_general(
            piscrow, term, (((1,), (0,)), ((), ())),
            preferred_element_type=f32)

    @pl.when(i == nb - 1)
    def _fin():
        out_ref[...] = sp_ref[...] + 0.1 * jnp.sum(acc_ref[...],
                                                   axis=1, keepdims=True)


def _sc_gather(weight, idx):
    dpad = 128
    info = plsc.get_sparse_core_info()
    nw = info.num_cores * info.num_subcores
    bpw = N_EMB // nw
    mesh = plsc.VectorSubcoreMesh(core_axis_name="c", subcore_axis_name="s")

    @functools.partial(
        pl.kernel, mesh=mesh,
        out_type=jax.ShapeDtypeStruct((N_EMB, dpad), jnp.float32),
        scratch_types=[
            pltpu.VMEM((bpw,), jnp.int32),
            pltpu.VMEM((bpw, dpad), jnp.float32),
            pltpu.SemaphoreType.DMA,
        ],
    )
    def gk(table_hbm, idx_hbm, out_hbm, idx_v, rows_v, sem):
        wid = lax.axis_index("s") * info.num_cores + lax.axis_index("c")
        base = wid * bpw
        pltpu.sync_copy(idx_hbm.at[pl.ds(base, bpw)], idx_v)
        pltpu.async_copy(table_hbm.at[idx_v], rows_v, sem).wait()
        pltpu.sync_copy(rows_v, out_hbm.at[pl.ds(base, bpw)])

    wp = jnp.pad(weight, ((0, 0), (0, dpad - E_DIM)))
    return gk(wp, idx)[:, :E_DIM]


def kernel(x, batch, weight, running_prior):
    n = x.shape[0] * x.shape[1]
    xf = x.reshape(n, E_DIM)
    rp = running_prior.reshape(1, N_EMB)

    f32 = jnp.float32
    w0n, q1t, q1r, q2a, pisc, idx2d, sp = pl.pallas_call(
        _prep_body,
        out_shape=[
            jax.ShapeDtypeStruct((N_EMB, n), jnp.bfloat16),
            jax.ShapeDtypeStruct((N_EMB, n), jnp.bfloat16),
            jax.ShapeDtypeStruct((N_EMB, n), jnp.bfloat16),
            jax.ShapeDtypeStruct((n, N_EMB), jnp.bfloat16),
            jax.ShapeDtypeStruct((N_EMB, N_EMB), f32),
            jax.ShapeDtypeStruct((n, 1), jnp.int32),
            jax.ShapeDtypeStruct((1, 1), f32),
        ],
    )(xf, weight, rp)

    quant = _sc_gather(weight, idx2d.reshape(n))

    nb = N_EMB // KB
    total = pl.pallas_call(
        _h2_body,
        grid=(nb,),
        in_specs=[
            pl.BlockSpec((KB, n), lambda i: (i, 0)),
            pl.BlockSpec((KB, N_EMB), lambda i: (i, 0)),
            pl.BlockSpec((N_EMB, n), lambda i: (0, 0)),
            pl.BlockSpec((N_EMB, n), lambda i: (0, 0)),
            pl.BlockSpec((n, N_EMB), lambda i: (0, 0)),
            pl.BlockSpec((1, 1), lambda i: (0, 0)),
        ],
        out_specs=pl.BlockSpec((1, 1), lambda i: (0, 0)),
        out_shape=jax.ShapeDtypeStruct((1, 1), f32),
        scratch_shapes=[pltpu.VMEM((1, N_EMB), f32)],
    )(w0n, pisc, q1t, q1r, q2a, sp)

    return quant.reshape(x.shape), total[0, 0]


# speedup vs baseline: 1.9481x; 1.0493x over previous
import functools

import jax
import jax.numpy as jnp
from jax import lax
from jax.experimental import pallas as pl
from jax.experimental.pallas import tpu as pltpu
from jax.experimental.pallas import tpu_sc as plsc

N_EMB = 512
E_DIM = 64
ALPHA = 0.001
EPS = 1e-8
KB = 8


def _prep_body(xf_ref, w_ref, rp_ref,
               w0n_ref, q1t_ref, q1r_ref, q2a_ref, pisc_ref, idx_ref, sp_ref):
    n = xf_ref.shape[0]
    seg = 256
    f32 = jnp.float32
    xf = xf_ref[...]
    w = w_ref[...]
    ones_row = jnp.ones((1, E_DIM), f32)

    x2c = jnp.sum(xf * xf, axis=1, keepdims=True)
    w2c = jnp.sum(w * w, axis=1, keepdims=True)
    x2r = lax.dot_general(ones_row, xf * xf, (((1,), (1,)), ((), ())),
                          preferred_element_type=f32)
    w2r = lax.dot_general(ones_row, w * w, (((1,), (1,)), ((), ())),
                          preferred_element_type=f32)
    mm = lax.dot_general(xf, w, (((1,), (1,)), ((), ())),
                         preferred_element_type=f32)
    mmt = lax.dot_general(w, xf, (((1,), (1,)), ((), ())),
                          preferred_element_type=f32)
    d = (x2c + w2r) - 2.0 * mm
    dt = (w2c + x2r) - 2.0 * mmt

    mn = jnp.min(d, axis=1, keepdims=True)
    iota_k = lax.broadcasted_iota(jnp.int32, (n, N_EMB), 1)
    cand = jnp.where(d == mn, iota_k, jnp.int32(N_EMB))
    idxc = jnp.min(cand, axis=1, keepdims=True)
    idx_ref[...] = idxc
    onehot = (iota_k == idxc).astype(f32)
    counts = jnp.sum(onehot, axis=0, keepdims=True)
    p = counts * (1.0 / n)
    rp = rp_ref[...]
    kl = jnp.sum(p * (jnp.log(p + 1e-10) - jnp.log(rp + 1e-10)))
    mse = jnp.sum(mn) * (1.0 / (n * E_DIM))
    sp_ref[...] = jnp.full((1, 1), 0.0, f32) + (1.25 * mse + 1.0 * kl)

    mx = jnp.max(d, axis=1, keepdims=True)
    sh = d - mx
    lse = jnp.log(jnp.sum(jnp.exp(sh), axis=1, keepdims=True))
    q = sh - lse
    mxr = jnp.max(dt, axis=0, keepdims=True)
    sht = dt - mxr
    lser = jnp.log(jnp.sum(jnp.exp(sht), axis=0, keepdims=True))
    qt = sht - lser

    lane_f = lax.broadcasted_iota(jnp.int32, (1, n), 1) % seg
    w0 = jnp.where(lane_f < seg - 2, qt, 0.0)
    q1t = jnp.concatenate([qt[:, 1:], qt[:, :1]], axis=1)
    q2s = jnp.concatenate([q[2:], q[:2]], axis=0)

    qpt = jnp.where(lane_f < seg - 1, qt, 0.0)
    cp = lax.dot_general(qpt, q1t, (((1,), (1,)), ((), ())),
                         preferred_element_type=f32)
    pi = cp * (1.0 / (jnp.sum(cp) + EPS))

    r2row = lax.dot_general(jnp.ones((1, n), f32), q2s,
                            (((1,), (1,)), ((), ())),
                            preferred_element_type=f32)
    s0km = lax.dot_general(w0 * r2row, q1t, (((1,), (1,)), ((), ())),
                           preferred_element_type=f32)
    dkm = s0km + (N_EMB * ALPHA + EPS)
    pisc_ref[...] = jnp.float32(0.6931471805599453) * pi / dkm

    bf16 = jnp.bfloat16
    w0n_ref[...] = (-w0).astype(bf16)
    q1t_ref[...] = q1t.astype(bf16)
    q1r_ref[...] = (q1t * r2row).astype(bf16)
    q2a_ref[...] = q2s.astype(bf16)


def _h2_body(w0n_ref, pisc_ref, q1t_ref, q1r_ref, q2a_ref, sp_ref,
             out_ref, acc_ref, acc2_ref):
    i = pl.program_id(0)
    nb = pl.num_programs(0)
    f32 = jnp.float32

    @pl.when(i == 0)
    def _init():
        acc_ref[...] = jnp.zeros((1, N_EMB), f32)
        acc2_ref[...] = jnp.zeros((1, 1), f32)

    q1t = q1t_ref[...]
    q2a = q2a_ref[...]
    w0n = w0n_ref[...]
    s0blk = lax.dot_general(q1r_ref[...], w0n, (((1,), (1,)), ((), ())),
                            preferred_element_type=f32)
    negdall = s0blk - (N_EMB * ALPHA + EPS)
    vall = -jnp.log2(negdall) * s0blk
    for kk in range(KB):
        wrow = w0n[kk:kk + 1, :]
        bmat = q1t * wrow
        ncs = lax.dot_general(bmat, q2a, (((1,), (0,)), ((), ())),
                              preferred_element_type=f32)
        e = ncs * jnp.log2(ncs)
        piscrow = pisc_ref[kk:kk + 1, :]
        acc_ref[...] += lax.dot_general(
            piscrow, e, (((1,), (0,)), ((), ())),
            preferred_element_type=f32)
        acc2_ref[...] += lax.dot_general(
            piscrow, vall[:, kk:kk + 1], (((1,), (0,)), ((), ())),
            preferred_element_type=f32)

    @pl.when(i == nb - 1)
    def _fin():
        out_ref[...] = sp_ref[...] + 0.1 * (
            jnp.sum(acc_ref[...], axis=1, keepdims=True) + acc2_ref[...])


def _sc_gather(weight, idx):
    dpad = 128
    info = plsc.get_sparse_core_info()
    nw = info.num_cores * info.num_subcores
    bpw = N_EMB // nw
    mesh = plsc.VectorSubcoreMesh(core_axis_name="c", subcore_axis_name="s")

    @functools.partial(
        pl.kernel, mesh=mesh,
        out_type=jax.ShapeDtypeStruct((N_EMB, dpad), jnp.float32),
        scratch_types=[
            pltpu.VMEM((bpw,), jnp.int32),
            pltpu.VMEM((bpw, dpad), jnp.float32),
            pltpu.SemaphoreType.DMA,
        ],
    )
    def gk(table_hbm, idx_hbm, out_hbm, idx_v, rows_v, sem):
        wid = lax.axis_index("s") * info.num_cores + lax.axis_index("c")
        base = wid * bpw
        pltpu.sync_copy(idx_hbm.at[pl.ds(base, bpw)], idx_v)
        pltpu.async_copy(table_hbm.at[idx_v], rows_v, sem).wait()
        pltpu.sync_copy(rows_v, out_hbm.at[pl.ds(base, bpw)])

    wp = jnp.pad(weight, ((0, 0), (0, dpad - E_DIM)))
    return gk(wp, idx)[:, :E_DIM]


def kernel(x, batch, weight, running_prior):
    n = x.shape[0] * x.shape[1]
    xf = x.reshape(n, E_DIM)
    rp = running_prior.reshape(1, N_EMB)

    f32 = jnp.float32
    w0n, q1t, q1r, q2a, pisc, idx2d, sp = pl.pallas_call(
        _prep_body,
        out_shape=[
            jax.ShapeDtypeStruct((N_EMB, n), jnp.bfloat16),
            jax.ShapeDtypeStruct((N_EMB, n), jnp.bfloat16),
            jax.ShapeDtypeStruct((N_EMB, n), jnp.bfloat16),
            jax.ShapeDtypeStruct((n, N_EMB), jnp.bfloat16),
            jax.ShapeDtypeStruct((N_EMB, N_EMB), f32),
            jax.ShapeDtypeStruct((n, 1), jnp.int32),
            jax.ShapeDtypeStruct((1, 1), f32),
        ],
    )(xf, weight, rp)

    quant = _sc_gather(weight, idx2d.reshape(n))

    nb = N_EMB // KB
    total = pl.pallas_call(
        _h2_body,
        grid=(nb,),
        in_specs=[
            pl.BlockSpec((KB, n), lambda i: (i, 0)),
            pl.BlockSpec((KB, N_EMB), lambda i: (i, 0)),
            pl.BlockSpec((N_EMB, n), lambda i: (0, 0)),
            pl.BlockSpec((N_EMB, n), lambda i: (0, 0)),
            pl.BlockSpec((n, N_EMB), lambda i: (0, 0)),
            pl.BlockSpec((1, 1), lambda i: (0, 0)),
        ],
        out_specs=pl.BlockSpec((1, 1), lambda i: (0, 0)),
        out_shape=jax.ShapeDtypeStruct((1, 1), f32),
        scratch_shapes=[pltpu.VMEM((1, N_EMB), f32), pltpu.VMEM((1, 1), f32)],
    )(w0n, pisc, q1t, q1r, q2a, sp)

    return quant.reshape(x.shape), total[0, 0]


# speedup vs baseline: 2.0083x; 1.0309x over previous
import functools

import jax
import jax.numpy as jnp
from jax import lax
from jax.experimental import pallas as pl
from jax.experimental.pallas import tpu as pltpu
from jax.experimental.pallas import tpu_sc as plsc

N_EMB = 512
E_DIM = 64
ALPHA = 0.001
EPS = 1e-8
KB = 16


def _prep_body(xf_ref, w_ref, rp_ref,
               w0n_ref, q1t_ref, q1r_ref, q2a_ref, pisc_ref, idx_ref, sp_ref):
    n = xf_ref.shape[0]
    seg = 256
    f32 = jnp.float32
    xf = xf_ref[...]
    w = w_ref[...]
    ones_row = jnp.ones((1, E_DIM), f32)

    x2c = jnp.sum(xf * xf, axis=1, keepdims=True)
    w2c = jnp.sum(w * w, axis=1, keepdims=True)
    x2r = lax.dot_general(ones_row, xf * xf, (((1,), (1,)), ((), ())),
                          preferred_element_type=f32)
    w2r = lax.dot_general(ones_row, w * w, (((1,), (1,)), ((), ())),
                          preferred_element_type=f32)
    mm = lax.dot_general(xf, w, (((1,), (1,)), ((), ())),
                         preferred_element_type=f32)
    mmt = lax.dot_general(w, xf, (((1,), (1,)), ((), ())),
                          preferred_element_type=f32)
    d = (x2c + w2r) - 2.0 * mm
    dt = (w2c + x2r) - 2.0 * mmt

    mn = jnp.min(d, axis=1, keepdims=True)
    iota_k = lax.broadcasted_iota(jnp.int32, (n, N_EMB), 1)
    cand = jnp.where(d == mn, iota_k, jnp.int32(N_EMB))
    idxc = jnp.min(cand, axis=1, keepdims=True)
    idx_ref[...] = idxc
    onehot = (iota_k == idxc).astype(f32)
    counts = jnp.sum(onehot, axis=0, keepdims=True)
    p = counts * (1.0 / n)
    rp = rp_ref[...]
    kl = jnp.sum(p * (jnp.log(p + 1e-10) - jnp.log(rp + 1e-10)))
    mse = jnp.sum(mn) * (1.0 / (n * E_DIM))
    sp_ref[...] = jnp.full((1, 1), 0.0, f32) + (1.25 * mse + 1.0 * kl)

    mx = jnp.max(d, axis=1, keepdims=True)
    sh = d - mx
    lse = jnp.log(jnp.sum(jnp.exp(sh), axis=1, keepdims=True))
    q = sh - lse
    mxr = jnp.max(dt, axis=0, keepdims=True)
    sht = dt - mxr
    lser = jnp.log(jnp.sum(jnp.exp(sht), axis=0, keepdims=True))
    qt = sht - lser

    lane_f = lax.broadcasted_iota(jnp.int32, (1, n), 1) % seg
    w0 = jnp.where(lane_f < seg - 2, qt, 0.0)
    q1t = jnp.concatenate([qt[:, 1:], qt[:, :1]], axis=1)
    q2s = jnp.concatenate([q[2:], q[:2]], axis=0)

    qpt = jnp.where(lane_f < seg - 1, qt, 0.0)
    cp = lax.dot_general(qpt, q1t, (((1,), (1,)), ((), ())),
                         preferred_element_type=f32)
    pi = cp * (1.0 / (jnp.sum(cp) + EPS))

    r2row = lax.dot_general(jnp.ones((1, n), f32), q2s,
                            (((1,), (1,)), ((), ())),
                            preferred_element_type=f32)
    s0km = lax.dot_general(w0 * r2row, q1t, (((1,), (1,)), ((), ())),
                           preferred_element_type=f32)
    dkm = s0km + (N_EMB * ALPHA + EPS)
    pisc_ref[...] = jnp.float32(0.6931471805599453) * pi / dkm

    bf16 = jnp.bfloat16
    w0n_ref[...] = (-w0).astype(bf16)
    q1t_ref[...] = q1t.astype(bf16)
    q1r_ref[...] = (q1t * r2row).astype(bf16)
    q2a_ref[...] = q2s.astype(bf16)


def _h2_body(w0n_ref, pisc_ref, q1t_ref, q1r_ref, q2a_ref, sp_ref,
             out_ref, acc_ref, acc2_ref):
    i = pl.program_id(0)
    nb = pl.num_programs(0)
    f32 = jnp.float32

    @pl.when(i == 0)
    def _init():
        acc_ref[...] = jnp.zeros((1, N_EMB), f32)
        acc2_ref[...] = jnp.zeros((1, 1), f32)

    q1t = q1t_ref[...]
    q2a = q2a_ref[...]
    w0n = w0n_ref[...]
    s0t = lax.dot_general(w0n, q1r_ref[...], (((1,), (1,)), ((), ())),
                          preferred_element_type=f32)
    negdt = s0t - (N_EMB * ALPHA + EPS)
    vallt = -jnp.log2(negdt) * s0t
    acc2_ref[...] += jnp.sum(pisc_ref[...] * vallt).reshape(1, 1)
    for kk in range(KB):
        wrow = w0n[kk:kk + 1, :]
        bmat = q1t * wrow
        ncs = lax.dot_general(bmat, q2a, (((1,), (0,)), ((), ())),
                              preferred_element_type=f32)
        e = ncs * jnp.log2(ncs)
        piscrow = pisc_ref[kk:kk + 1, :]
        acc_ref[...] += lax.dot_general(
            piscrow, e, (((1,), (0,)), ((), ())),
            preferred_element_type=f32)

    @pl.when(i == nb - 1)
    def _fin():
        out_ref[...] = sp_ref[...] + 0.1 * (
            jnp.sum(acc_ref[...], axis=1, keepdims=True) + acc2_ref[...])


def _sc_gather(weight, idx):
    dpad = 128
    info = plsc.get_sparse_core_info()
    nw = info.num_cores * info.num_subcores
    bpw = N_EMB // nw
    mesh = plsc.VectorSubcoreMesh(core_axis_name="c", subcore_axis_name="s")

    @functools.partial(
        pl.kernel, mesh=mesh,
        out_type=jax.ShapeDtypeStruct((N_EMB, dpad), jnp.float32),
        scratch_types=[
            pltpu.VMEM((bpw,), jnp.int32),
            pltpu.VMEM((bpw, dpad), jnp.float32),
            pltpu.SemaphoreType.DMA,
        ],
    )
    def gk(table_hbm, idx_hbm, out_hbm, idx_v, rows_v, sem):
        wid = lax.axis_index("s") * info.num_cores + lax.axis_index("c")
        base = wid * bpw
        pltpu.sync_copy(idx_hbm.at[pl.ds(base, bpw)], idx_v)
        pltpu.async_copy(table_hbm.at[idx_v], rows_v, sem).wait()
        pltpu.sync_copy(rows_v, out_hbm.at[pl.ds(base, bpw)])

    wp = jnp.pad(weight, ((0, 0), (0, dpad - E_DIM)))
    return gk(wp, idx)[:, :E_DIM]


def kernel(x, batch, weight, running_prior):
    n = x.shape[0] * x.shape[1]
    xf = x.reshape(n, E_DIM)
    rp = running_prior.reshape(1, N_EMB)

    f32 = jnp.float32
    w0n, q1t, q1r, q2a, pisc, idx2d, sp = pl.pallas_call(
        _prep_body,
        out_shape=[
            jax.ShapeDtypeStruct((N_EMB, n), jnp.bfloat16),
            jax.ShapeDtypeStruct((N_EMB, n), jnp.bfloat16),
            jax.ShapeDtypeStruct((N_EMB, n), jnp.bfloat16),
            jax.ShapeDtypeStruct((n, N_EMB), jnp.bfloat16),
            jax.ShapeDtypeStruct((N_EMB, N_EMB), f32),
            jax.ShapeDtypeStruct((n, 1), jnp.int32),
            jax.ShapeDtypeStruct((1, 1), f32),
        ],
    )(xf, weight, rp)

    quant = _sc_gather(weight, idx2d.reshape(n))

    nb = N_EMB // KB
    total = pl.pallas_call(
        _h2_body,
        grid=(nb,),
        in_specs=[
            pl.BlockSpec((KB, n), lambda i: (i, 0)),
            pl.BlockSpec((KB, N_EMB), lambda i: (i, 0)),
            pl.BlockSpec((N_EMB, n), lambda i: (0, 0)),
            pl.BlockSpec((N_EMB, n), lambda i: (0, 0)),
            pl.BlockSpec((n, N_EMB), lambda i: (0, 0)),
            pl.BlockSpec((1, 1), lambda i: (0, 0)),
        ],
        out_specs=pl.BlockSpec((1, 1), lambda i: (0, 0)),
        out_shape=jax.ShapeDtypeStruct((1, 1), f32),
        scratch_shapes=[pltpu.VMEM((1, N_EMB), f32), pltpu.VMEM((1, 1), f32)],
    )(w0n, pisc, q1t, q1r, q2a, sp)

    return quant.reshape(x.shape), total[0, 0]


# speedup vs baseline: 2.0185x; 1.0051x over previous
import functools

import jax
import jax.numpy as jnp
from jax import lax
from jax.experimental import pallas as pl
from jax.experimental.pallas import tpu as pltpu
from jax.experimental.pallas import tpu_sc as plsc

N_EMB = 512
E_DIM = 64
ALPHA = 0.001
EPS = 1e-8
KB = 16


def _prep_body(xf_ref, w_ref, rp_ref,
               w0n_ref, q1t_ref, q1r_ref, q2a_ref, pisc_ref, idx_ref, sp_ref):
    n = xf_ref.shape[0]
    seg = 256
    f32 = jnp.float32
    xf = xf_ref[...]
    w = w_ref[...]
    ones_row = jnp.ones((1, E_DIM), f32)

    x2c = jnp.sum(xf * xf, axis=1, keepdims=True)
    w2c = jnp.sum(w * w, axis=1, keepdims=True)
    x2r = lax.dot_general(ones_row, xf * xf, (((1,), (1,)), ((), ())),
                          preferred_element_type=f32)
    w2r = lax.dot_general(ones_row, w * w, (((1,), (1,)), ((), ())),
                          preferred_element_type=f32)
    mm = lax.dot_general(xf, w, (((1,), (1,)), ((), ())),
                         preferred_element_type=f32)
    mmt = lax.dot_general(w, xf, (((1,), (1,)), ((), ())),
                          preferred_element_type=f32)
    d = (x2c + w2r) - 2.0 * mm
    dt = (w2c + x2r) - 2.0 * mmt

    mn = jnp.min(d, axis=1, keepdims=True)
    iota_k = lax.broadcasted_iota(jnp.int32, (n, N_EMB), 1)
    cand = jnp.where(d == mn, iota_k, jnp.int32(N_EMB))
    idxc = jnp.min(cand, axis=1, keepdims=True)
    idx_ref[...] = idxc
    onehot = (iota_k == idxc).astype(f32)
    counts = jnp.sum(onehot, axis=0, keepdims=True)
    p = counts * (1.0 / n)
    rp = rp_ref[...]
    kl = jnp.sum(p * (jnp.log(p + 1e-10) - jnp.log(rp + 1e-10)))
    mse = jnp.sum(mn) * (1.0 / (n * E_DIM))
    sp_ref[...] = jnp.full((1, 1), 0.0, f32) + (1.25 * mse + 1.0 * kl)

    mx = jnp.max(d, axis=1, keepdims=True)
    sh = d - mx
    lse = jnp.log(jnp.sum(jnp.exp(sh), axis=1, keepdims=True))
    q = sh - lse
    mxr = jnp.max(dt, axis=0, keepdims=True)
    sht = dt - mxr
    lser = jnp.log(jnp.sum(jnp.exp(sht), axis=0, keepdims=True))
    qt = sht - lser

    lane_f = lax.broadcasted_iota(jnp.int32, (1, n), 1) % seg
    w0 = jnp.where(lane_f < seg - 2, qt, 0.0)
    q1t = jnp.concatenate([qt[:, 1:], qt[:, :1]], axis=1)
    q2s = jnp.concatenate([q[2:], q[:2]], axis=0)

    qpt = jnp.where(lane_f < seg - 1, qt, 0.0)
    cp = lax.dot_general(qpt, q1t, (((1,), (1,)), ((), ())),
                         preferred_element_type=f32)
    pi = cp * (1.0 / (jnp.sum(cp) + EPS))

    r2row = lax.dot_general(jnp.ones((1, n), f32), q2s,
                            (((1,), (1,)), ((), ())),
                            preferred_element_type=f32)
    s0km = lax.dot_general(w0 * r2row, q1t, (((1,), (1,)), ((), ())),
                           preferred_element_type=f32)
    dkm = s0km + (N_EMB * ALPHA + EPS)
    pisc_ref[...] = jnp.float32(0.6931471805599453) * pi / dkm

    bf16 = jnp.bfloat16
    w0n_ref[...] = (-w0).astype(bf16)
    q1t_ref[...] = q1t.astype(bf16)
    q1r_ref[...] = (q1t * r2row).astype(bf16)
    q2a_ref[...] = q2s.astype(bf16)


def _h2_body(w0n_ref, pisc_ref, q1t_ref, q1r_ref, q2a_ref, sp_ref,
             out_ref, acc_ref, acc2_ref):
    i = pl.program_id(0)
    nb = pl.num_programs(0)
    f32 = jnp.float32

    @pl.when(i == 0)
    def _init():
        acc_ref[...] = jnp.zeros((1, N_EMB), f32)
        acc2_ref[...] = jnp.zeros((1, 1), f32)

    q1t = q1t_ref[...]
    q2a = q2a_ref[...]
    w0n = w0n_ref[...]
    s0t = lax.dot_general(w0n, q1r_ref[...], (((1,), (1,)), ((), ())),
                          preferred_element_type=f32)
    negdt = s0t - (N_EMB * ALPHA + EPS)
    vallt = -jnp.log2(negdt) * s0t
    acc2_ref[...] += jnp.sum(pisc_ref[...] * vallt).reshape(1, 1)
    for kk in range(KB):
        wrow = w0n[kk:kk + 1, :]
        bmat = q1t * wrow
        ncs = lax.dot_general(bmat, q2a, (((1,), (0,)), ((), ())),
                              preferred_element_type=f32)
        e = ncs * jnp.log2(ncs)
        pisc8 = jnp.where(
            lax.broadcasted_iota(jnp.int32, (8, N_EMB), 0) == 0,
            pisc_ref[kk:kk + 1, :], 0.0)
        acc_ref[...] += lax.dot_general(
            pisc8, e, (((1,), (0,)), ((), ())),
            preferred_element_type=f32)[0:1, :]

    @pl.when(i == nb - 1)
    def _fin():
        out_ref[...] = sp_ref[...] + 0.1 * (
            jnp.sum(acc_ref[...], axis=1, keepdims=True) + acc2_ref[...])


def _sc_gather(weight, idx):
    dpad = 128
    info = plsc.get_sparse_core_info()
    nw = info.num_cores * info.num_subcores
    bpw = N_EMB // nw
    mesh = plsc.VectorSubcoreMesh(core_axis_name="c", subcore_axis_name="s")

    @functools.partial(
        pl.kernel, mesh=mesh,
        out_type=jax.ShapeDtypeStruct((N_EMB, dpad), jnp.float32),
        scratch_types=[
            pltpu.VMEM((bpw,), jnp.int32),
            pltpu.VMEM((bpw, dpad), jnp.float32),
            pltpu.SemaphoreType.DMA,
        ],
    )
    def gk(table_hbm, idx_hbm, out_hbm, idx_v, rows_v, sem):
        wid = lax.axis_index("s") * info.num_cores + lax.axis_index("c")
        base = wid * bpw
        pltpu.sync_copy(idx_hbm.at[pl.ds(base, bpw)], idx_v)
        pltpu.async_copy(table_hbm.at[idx_v], rows_v, sem).wait()
        pltpu.sync_copy(rows_v, out_hbm.at[pl.ds(base, bpw)])

    wp = jnp.pad(weight, ((0, 0), (0, dpad - E_DIM)))
    return gk(wp, idx)[:, :E_DIM]


def kernel(x, batch, weight, running_prior):
    n = x.shape[0] * x.shape[1]
    xf = x.reshape(n, E_DIM)
    rp = running_prior.reshape(1, N_EMB)

    f32 = jnp.float32
    w0n, q1t, q1r, q2a, pisc, idx2d, sp = pl.pallas_call(
        _prep_body,
        out_shape=[
            jax.ShapeDtypeStruct((N_EMB, n), jnp.bfloat16),
            jax.ShapeDtypeStruct((N_EMB, n), jnp.bfloat16),
            jax.ShapeDtypeStruct((N_EMB, n), jnp.bfloat16),
            jax.ShapeDtypeStruct((n, N_EMB), jnp.bfloat16),
            jax.ShapeDtypeStruct((N_EMB, N_EMB), f32),
            jax.ShapeDtypeStruct((n, 1), jnp.int32),
            jax.ShapeDtypeStruct((1, 1), f32),
        ],
    )(xf, weight, rp)

    quant = _sc_gather(weight, idx2d.reshape(n))

    nb = N_EMB // KB
    total = pl.pallas_call(
        _h2_body,
        grid=(nb,),
        in_specs=[
            pl.BlockSpec((KB, n), lambda i: (i, 0)),
            pl.BlockSpec((KB, N_EMB), lambda i: (i, 0)),
            pl.BlockSpec((N_EMB, n), lambda i: (0, 0)),
            pl.BlockSpec((N_EMB, n), lambda i: (0, 0)),
            pl.BlockSpec((n, N_EMB), lambda i: (0, 0)),
            pl.BlockSpec((1, 1), lambda i: (0, 0)),
        ],
        out_specs=pl.BlockSpec((1, 1), lambda i: (0, 0)),
        out_shape=jax.ShapeDtypeStruct((1, 1), f32),
        scratch_shapes=[pltpu.VMEM((1, N_EMB), f32), pltpu.VMEM((1, 1), f32)],
    )(w0n, pisc, q1t, q1r, q2a, sp)

    return quant.reshape(x.shape), total[0, 0]


# speedup vs baseline: 2.0529x; 1.0170x over previous
import functools

import jax
import jax.numpy as jnp
from jax import lax
from jax.experimental import pallas as pl
from jax.experimental.pallas import tpu as pltpu
from jax.experimental.pallas import tpu_sc as plsc

N_EMB = 512
E_DIM = 64
ALPHA = 0.001
EPS = 1e-8
KB = 32


def _prep_body(xf_ref, w_ref, rp_ref,
               w0n_ref, q1t_ref, q1r_ref, q2a_ref, pisc_ref, idx_ref, sp_ref):
    n = xf_ref.shape[0]
    seg = 256
    f32 = jnp.float32
    xf = xf_ref[...]
    w = w_ref[...]
    ones_row = jnp.ones((1, E_DIM), f32)

    x2c = jnp.sum(xf * xf, axis=1, keepdims=True)
    w2c = jnp.sum(w * w, axis=1, keepdims=True)
    x2r = lax.dot_general(ones_row, xf * xf, (((1,), (1,)), ((), ())),
                          preferred_element_type=f32)
    w2r = lax.dot_general(ones_row, w * w, (((1,), (1,)), ((), ())),
                          preferred_element_type=f32)
    mm = lax.dot_general(xf, w, (((1,), (1,)), ((), ())),
                         preferred_element_type=f32)
    mmt = lax.dot_general(w, xf, (((1,), (1,)), ((), ())),
                          preferred_element_type=f32)
    d = (x2c + w2r) - 2.0 * mm
    dt = (w2c + x2r) - 2.0 * mmt

    mn = jnp.min(d, axis=1, keepdims=True)
    iota_k = lax.broadcasted_iota(jnp.int32, (n, N_EMB), 1)
    cand = jnp.where(d == mn, iota_k, jnp.int32(N_EMB))
    idxc = jnp.min(cand, axis=1, keepdims=True)
    idx_ref[...] = idxc
    onehot = (iota_k == idxc).astype(f32)
    counts = jnp.sum(onehot, axis=0, keepdims=True)
    p = counts * (1.0 / n)
    rp = rp_ref[...]
    kl = jnp.sum(p * (jnp.log(p + 1e-10) - jnp.log(rp + 1e-10)))
    mse = jnp.sum(mn) * (1.0 / (n * E_DIM))
    sp_ref[...] = jnp.full((1, 1), 0.0, f32) + (1.25 * mse + 1.0 * kl)

    mx = jnp.max(d, axis=1, keepdims=True)
    sh = d - mx
    lse = jnp.log(jnp.sum(jnp.exp(sh), axis=1, keepdims=True))
    q = sh - lse
    mxr = jnp.max(dt, axis=0, keepdims=True)
    sht = dt - mxr
    lser = jnp.log(jnp.sum(jnp.exp(sht), axis=0, keepdims=True))
    qt = sht - lser

    lane_f = lax.broadcasted_iota(jnp.int32, (1, n), 1) % seg
    w0 = jnp.where(lane_f < seg - 2, qt, 0.0)
    q1t = jnp.concatenate([qt[:, 1:], qt[:, :1]], axis=1)
    q2s = jnp.concatenate([q[2:], q[:2]], axis=0)

    qpt = jnp.where(lane_f < seg - 1, qt, 0.0)
    cp = lax.dot_general(qpt, q1t, (((1,), (1,)), ((), ())),
                         preferred_element_type=f32)
    pi = cp * (1.0 / (jnp.sum(cp) + EPS))

    r2row = lax.dot_general(jnp.ones((1, n), f32), q2s,
                            (((1,), (1,)), ((), ())),
                            preferred_element_type=f32)
    s0km = lax.dot_general(w0 * r2row, q1t, (((1,), (1,)), ((), ())),
                           preferred_element_type=f32)
    dkm = s0km + (N_EMB * ALPHA + EPS)
    pisc_ref[...] = jnp.float32(0.6931471805599453) * pi / dkm

    bf16 = jnp.bfloat16
    w0n_ref[...] = (-w0).astype(bf16)
    q1t_ref[...] = q1t.astype(bf16)
    q1r_ref[...] = (q1t * r2row).astype(bf16)
    q2a_ref[...] = q2s.astype(bf16)


def _h2_body(w0n_ref, pisc_ref, q1t_ref, q1r_ref, q2a_ref, sp_ref,
             out_ref, acc_ref, acc2_ref):
    i = pl.program_id(0)
    nb = pl.num_programs(0)
    f32 = jnp.float32

    @pl.when(i == 0)
    def _init():
        acc_ref[...] = jnp.zeros((1, N_EMB), f32)
        acc2_ref[...] = jnp.zeros((1, 1), f32)

    q1t = q1t_ref[...]
    q2a = q2a_ref[...]
    w0n = w0n_ref[...]
    s0t = lax.dot_general(w0n, q1r_ref[...], (((1,), (1,)), ((), ())),
                          preferred_element_type=f32)
    negdt = s0t - (N_EMB * ALPHA + EPS)
    vallt = -jnp.log2(negdt) * s0t
    acc2_ref[...] += jnp.sum(pisc_ref[...] * vallt).reshape(1, 1)
    for kk in range(KB):
        wrow = w0n[kk:kk + 1, :]
        bmat = q1t * wrow
        ncs = lax.dot_general(bmat, q2a, (((1,), (0,)), ((), ())),
                              preferred_element_type=f32)
        e = ncs * jnp.log2(ncs)
        pisc8 = jnp.where(
            lax.broadcasted_iota(jnp.int32, (8, N_EMB), 0) == 0,
            pisc_ref[kk:kk + 1, :], 0.0)
        acc_ref[...] += lax.dot_general(
            pisc8, e, (((1,), (0,)), ((), ())),
            preferred_element_type=f32)[0:1, :]

    @pl.when(i == nb - 1)
    def _fin():
        out_ref[...] = sp_ref[...] + 0.1 * (
            jnp.sum(acc_ref[...], axis=1, keepdims=True) + acc2_ref[...])


def _sc_gather(weight, idx):
    dpad = 128
    info = plsc.get_sparse_core_info()
    nw = info.num_cores * info.num_subcores
    bpw = N_EMB // nw
    mesh = plsc.VectorSubcoreMesh(core_axis_name="c", subcore_axis_name="s")

    @functools.partial(
        pl.kernel, mesh=mesh,
        out_type=jax.ShapeDtypeStruct((N_EMB, dpad), jnp.float32),
        scratch_types=[
            pltpu.VMEM((bpw,), jnp.int32),
            pltpu.VMEM((bpw, dpad), jnp.float32),
            pltpu.SemaphoreType.DMA,
        ],
    )
    def gk(table_hbm, idx_hbm, out_hbm, idx_v, rows_v, sem):
        wid = lax.axis_index("s") * info.num_cores + lax.axis_index("c")
        base = wid * bpw
        pltpu.sync_copy(idx_hbm.at[pl.ds(base, bpw)], idx_v)
        pltpu.async_copy(table_hbm.at[idx_v], rows_v, sem).wait()
        pltpu.sync_copy(rows_v, out_hbm.at[pl.ds(base, bpw)])

    wp = jnp.pad(weight, ((0, 0), (0, dpad - E_DIM)))
    return gk(wp, idx)[:, :E_DIM]


def kernel(x, batch, weight, running_prior):
    n = x.shape[0] * x.shape[1]
    xf = x.reshape(n, E_DIM)
    rp = running_prior.reshape(1, N_EMB)

    f32 = jnp.float32
    w0n, q1t, q1r, q2a, pisc, idx2d, sp = pl.pallas_call(
        _prep_body,
        out_shape=[
            jax.ShapeDtypeStruct((N_EMB, n), jnp.bfloat16),
            jax.ShapeDtypeStruct((N_EMB, n), jnp.bfloat16),
            jax.ShapeDtypeStruct((N_EMB, n), jnp.bfloat16),
            jax.ShapeDtypeStruct((n, N_EMB), jnp.bfloat16),
            jax.ShapeDtypeStruct((N_EMB, N_EMB), f32),
            jax.ShapeDtypeStruct((n, 1), jnp.int32),
            jax.ShapeDtypeStruct((1, 1), f32),
        ],
    )(xf, weight, rp)

    quant = _sc_gather(weight, idx2d.reshape(n))

    nb = N_EMB // KB
    total = pl.pallas_call(
        _h2_body,
        grid=(nb,),
        in_specs=[
            pl.BlockSpec((KB, n), lambda i: (i, 0)),
            pl.BlockSpec((KB, N_EMB), lambda i: (i, 0)),
            pl.BlockSpec((N_EMB, n), lambda i: (0, 0)),
            pl.BlockSpec((N_EMB, n), lambda i: (0, 0)),
            pl.BlockSpec((n, N_EMB), lambda i: (0, 0)),
            pl.BlockSpec((1, 1), lambda i: (0, 0)),
        ],
        out_specs=pl.BlockSpec((1, 1), lambda i: (0, 0)),
        out_shape=jax.ShapeDtypeStruct((1, 1), f32),
        scratch_shapes=[pltpu.VMEM((1, N_EMB), f32), pltpu.VMEM((1, 1), f32)],
    )(w0n, pisc, q1t, q1r, q2a, sp)

    return quant.reshape(x.shape), total[0, 0]
